# baseline (speedup 1.0000x reference)
import jax
import jax.numpy as jnp
from jax import lax
from jax.experimental import pallas as pl
from jax.experimental.pallas import tpu as pltpu
from jax.experimental.pallas import tpu_sc as plsc

VOCAB = 1000
VPAD = 1024
EMB = 300
NCLS = 20
NPAIR = NCLS // 2
B = 4096
L = 50

NC, NS = 2, 16
NW = NC * NS
DOCS_PER_W = B // NW
GROUPS = DOCS_PER_W // 16


def _tc_project(tablet_ref, w_ref, b_ref, mtp_ref, bias_ref):
    mt = lax.dot_general(
        w_ref[...], tablet_ref[...],
        (((1,), (0,)), ((), ())),
        preferred_element_type=jnp.float32,
    ) * (1.0 / L)
    lo = lax.convert_element_type(mt[:NPAIR, :], jnp.bfloat16)
    hi = lax.convert_element_type(mt[NPAIR:, :], jnp.bfloat16)
    lo_u = lax.convert_element_type(
        lax.bitcast_convert_type(lo, jnp.uint16), jnp.uint32)
    hi_u = lax.convert_element_type(
        lax.bitcast_convert_type(hi, jnp.uint16), jnp.uint32)
    packed = lax.bitcast_convert_type(lo_u | (hi_u << 16), jnp.int32)
    mtp_ref[:, :VOCAB] = packed
    mtp_ref[:, VOCAB:] = jnp.zeros((NPAIR, VPAD - VOCAB), jnp.int32)
    bias_ref[...] = jnp.broadcast_to(b_ref[...], (16, NCLS))


def _sc_pool(mtp_hbm, bias_hbm, docst_hbm, outt_hbm, bias_v, docs_v,
             out_v, sem, *m_vs):
    cid = lax.axis_index("c")
    sid = lax.axis_index("s")
    wid = sid * NC + cid
    col0 = wid * DOCS_PER_W
    cps = [pltpu.async_copy(docst_hbm.at[:, pl.ds(col0, DOCS_PER_W)], docs_v,
                            sem),
           pltpu.async_copy(bias_hbm, bias_v, sem)]
    cps += [pltpu.async_copy(mtp_hbm.at[p], m_vs[p], sem)
            for p in range(NPAIR)]
    for cp in cps:
        cp.wait()
    lane = lax.iota(jnp.int32, 16)
    cls_idx = [jnp.full((16,), c, jnp.int32) for c in range(NCLS)]
    himask = jnp.full((16,), -65536, jnp.int32)

    def unpack2(w):
        a = plsc.bitcast(lax.shift_left(w, 16), jnp.float32)
        b = plsc.bitcast(w & himask, jnp.float32)
        return a, b

    for g in range(GROUPS):
        acc0 = tuple(plsc.load_gather(bias_v, [lane, cls_idx[c]])
                     for c in range(NCLS))

        @plsc.parallel_loop(0, L // 2, carry=acc0, unroll=2)
        def accs(l, accs, g=g):
            tok0 = docs_v[2 * l, pl.ds(g * 16, 16)]
            tok1 = docs_v[2 * l + 1, pl.ds(g * 16, 16)]
            lo_hi = []
            for p in range(NPAIR):
                a0, b0 = unpack2(plsc.load_gather(m_vs[p], [tok0]))
                a1, b1 = unpack2(plsc.load_gather(m_vs[p], [tok1]))
                lo_hi.append((a0 + a1, b0 + b1))
            return tuple(accs[p] + lo_hi[p][0] for p in range(NPAIR)) + \
                tuple(accs[NPAIR + p] + lo_hi[p][1] for p in range(NPAIR))
        for c in range(NCLS):
            out_v[c, pl.ds(g * 16, 16)] = accs[c]
    pltpu.sync_copy(out_v, outt_hbm.at[:, pl.ds(col0, DOCS_PER_W)])


def kernel(docs, table, W, b):
    mtp, bias16 = pl.pallas_call(
        _tc_project,
        out_shape=(
            jax.ShapeDtypeStruct((NPAIR, VPAD), jnp.int32),
            jax.ShapeDtypeStruct((16, NCLS), jnp.float32),
        ),
    )(table.T, W, b.reshape(1, NCLS))

    mesh = plsc.VectorSubcoreMesh(core_axis_name="c", subcore_axis_name="s",
                                  num_cores=NC, num_subcores=NS)
    sc = pl.kernel(
        _sc_pool,
        out_type=jax.ShapeDtypeStruct((NCLS, B), jnp.float32),
        mesh=mesh,
        compiler_params=pltpu.CompilerParams(needs_layout_passes=False),
        scratch_types=[
            pltpu.VMEM((16, NCLS), jnp.float32),
            pltpu.VMEM((L, DOCS_PER_W), jnp.int32),
            pltpu.VMEM((NCLS, DOCS_PER_W), jnp.float32),
            pltpu.SemaphoreType.DMA,
        ] + [pltpu.VMEM((VPAD,), jnp.int32) for _ in range(NPAIR)],
    )
    out_t = sc(mtp, bias16, docs.T)
    return out_t.T


# speedup vs baseline: 1.1207x; 1.1207x over previous
import jax
import jax.numpy as jnp
from jax import lax
from jax.experimental import pallas as pl
from jax.experimental.pallas import tpu as pltpu
from jax.experimental.pallas import tpu_sc as plsc

VOCAB = 1000
VPAD = 1024
EMB = 300
NCLS = 20
NPAIR = NCLS // 2
B = 4096
L = 50

NC, NS = 2, 16
NW = NC * NS
DOCS_PER_W = B // NW
GROUPS = DOCS_PER_W // 16


def _tc_project(tablet_ref, w_ref, b_ref, mtp_ref, bias_ref):
    mt = lax.dot_general(
        w_ref[...], tablet_ref[...],
        (((1,), (0,)), ((), ())),
        preferred_element_type=jnp.float32,
    ) * (1.0 / L)
    lo = lax.convert_element_type(mt[:NPAIR, :], jnp.bfloat16)
    hi = lax.convert_element_type(mt[NPAIR:, :], jnp.bfloat16)
    lo_u = lax.convert_element_type(
        lax.bitcast_convert_type(lo, jnp.uint16), jnp.uint32)
    hi_u = lax.convert_element_type(
        lax.bitcast_convert_type(hi, jnp.uint16), jnp.uint32)
    packed = lax.bitcast_convert_type(lo_u | (hi_u << 16), jnp.int32)
    mtp_ref[:, :VOCAB] = packed
    mtp_ref[:, VOCAB:] = jnp.zeros((NPAIR, VPAD - VOCAB), jnp.int32)
    bias_ref[...] = jnp.broadcast_to(b_ref[...], (16, NCLS))


def _sc_pool(mtp_hbm, bias_hbm, docst_hbm, outt_hbm, bias_v, docs_v,
             out_v, sem, *m_vs):
    cid = lax.axis_index("c")
    sid = lax.axis_index("s")
    wid = sid * NC + cid
    col0 = wid * DOCS_PER_W
    cps = [pltpu.async_copy(docst_hbm.at[:, pl.ds(col0, DOCS_PER_W)], docs_v,
                            sem),
           pltpu.async_copy(bias_hbm, bias_v, sem)]
    cps += [pltpu.async_copy(mtp_hbm.at[p], m_vs[p], sem)
            for p in range(NPAIR)]
    for cp in cps:
        cp.wait()
    lane = lax.iota(jnp.int32, 16)
    cls_idx = [jnp.full((16,), c, jnp.int32) for c in range(NCLS)]
    himask = jnp.full((16,), -65536, jnp.int32)

    def unpack2(w):
        a = plsc.bitcast(lax.shift_left(w, 16), jnp.float32)
        b = plsc.bitcast(w & himask, jnp.float32)
        return a, b

    for g in range(GROUPS):
        acc0 = tuple(plsc.load_gather(bias_v, [lane, cls_idx[c]])
                     for c in range(NCLS))

        @plsc.parallel_loop(0, L // 2, carry=acc0)
        def accs(l, accs, g=g):
            tok0 = docs_v[2 * l, pl.ds(g * 16, 16)]
            tok1 = docs_v[2 * l + 1, pl.ds(g * 16, 16)]
            lo_hi = []
            for p in range(NPAIR):
                a0, b0 = unpack2(plsc.load_gather(m_vs[p], [tok0]))
                a1, b1 = unpack2(plsc.load_gather(m_vs[p], [tok1]))
                lo_hi.append((a0 + a1, b0 + b1))
            return tuple(accs[p] + lo_hi[p][0] for p in range(NPAIR)) + \
                tuple(accs[NPAIR + p] + lo_hi[p][1] for p in range(NPAIR))
        for c in range(NCLS):
            out_v[c, pl.ds(g * 16, 16)] = accs[c]
    pltpu.sync_copy(out_v, outt_hbm.at[:, pl.ds(col0, DOCS_PER_W)])


def kernel(docs, table, W, b):
    mtp, bias16 = pl.pallas_call(
        _tc_project,
        out_shape=(
            jax.ShapeDtypeStruct((NPAIR, VPAD), jnp.int32),
            jax.ShapeDtypeStruct((16, NCLS), jnp.float32),
        ),
    )(table.T, W, b.reshape(1, NCLS))

    mesh = plsc.VectorSubcoreMesh(core_axis_name="c", subcore_axis_name="s",
                                  num_cores=NC, num_subcores=NS)
    sc = pl.kernel(
        _sc_pool,
        out_type=jax.ShapeDtypeStruct((NCLS, B), jnp.float32),
        mesh=mesh,
        compiler_params=pltpu.CompilerParams(needs_layout_passes=False),
        scratch_types=[
            pltpu.VMEM((16, NCLS), jnp.float32),
            pltpu.VMEM((L, DOCS_PER_W), jnp.int32),
            pltpu.VMEM((NCLS, DOCS_PER_W), jnp.float32),
            pltpu.SemaphoreType.DMA,
        ] + [pltpu.VMEM((VPAD,), jnp.int32) for _ in range(NPAIR)],
    )
    out_t = sc(mtp, bias16, docs.T)
    return out_t.T
